# baseline (speedup 1.0000x reference)
import functools

import jax
import jax.numpy as jnp
from jax import lax
from jax.experimental import pallas as pl
from jax.experimental.pallas import tpu as pltpu
from jax.experimental.pallas import tpu_sc as plsc

B = 16384
F = 26
FIELD = 100000
VPAD = 2600960
NW = 32
BPW = B // NW
E = BPW * F
L = 16
NC = 4
CS = BPW // NC
CHUNK = F * CS

_mesh = plsc.VectorSubcoreMesh(core_axis_name="c", subcore_axis_name="s")


@functools.partial(
    pl.kernel,
    mesh=_mesh,
    out_type=jax.ShapeDtypeStruct((B * F,), jnp.int32),
    scratch_types=[
        pltpu.VMEM((NC, F, CS), jnp.int32),
        pltpu.VMEM((NC, CHUNK), jnp.int32),
        pltpu.SemaphoreType.DMA((NC,)),
    ],
)
def _build_indices_sc(xt_hbm, idx_hbm, xv, idxv, xsem):
    wid = lax.axis_index("s") * 2 + lax.axis_index("c")
    base = wid * BPW

    for c in range(NC):
        pltpu.async_copy(xt_hbm.at[:, pl.ds(base + c * CS, CS)],
                         xv.at[c], xsem.at[c])

    for c in range(NC):
        pltpu.make_async_copy(xt_hbm.at[:, pl.ds(base + c * CS, CS)],
                              xv.at[c], xsem.at[c]).wait()

        def build(i, _, c=c):
            f = i // (CS // L)
            jj = (i % (CS // L)) * L
            idxv[c, pl.ds(i * L, L)] = xv[c, f, pl.ds(jj, L)] + f * FIELD
            return 0

        lax.fori_loop(0, CHUNK // L, build, 0, unroll=4)
        pltpu.sync_copy(idxv.at[c],
                        idx_hbm.at[pl.ds(wid * E + c * CHUNK, CHUNK)])


@functools.partial(
    pl.kernel,
    mesh=_mesh,
    out_type=jax.ShapeDtypeStruct((B,), jnp.float32),
    scratch_types=[
        *[pltpu.VMEM((CHUNK,), jnp.int32) for _ in range(NC)],
        *[pltpu.VMEM((CHUNK,), jnp.float32) for _ in range(NC)],
        pltpu.VMEM((BPW,), jnp.float32),
        pltpu.VMEM((L,), jnp.float32),
        pltpu.SemaphoreType.DMA((NC,)),
        pltpu.SemaphoreType.DMA((NC,)),
    ],
)
def _gather_reduce_sc(idx_hbm, bias_hbm, tab_hbm, out_hbm,
                      i0, i1, i2, i3, v0, v1, v2, v3,
                      outv, biasv, isem, gsem):
    idxs = (i0, i1, i2, i3)
    vals = (v0, v1, v2, v3)
    wid = lax.axis_index("s") * 2 + lax.axis_index("c")
    base = wid * BPW

    for c in range(NC):
        pltpu.async_copy(idx_hbm.at[pl.ds(wid * E + c * CHUNK, CHUNK)],
                         idxs[c], isem.at[c])

    for c in range(NC):
        pltpu.make_async_copy(idx_hbm.at[pl.ds(wid * E + c * CHUNK, CHUNK)],
                              idxs[c], isem.at[c]).wait()
        pltpu.async_copy(tab_hbm.at[idxs[c]], vals[c], gsem.at[c])

    pltpu.sync_copy(bias_hbm, biasv)

    for c in range(NC):
        pltpu.make_async_copy(tab_hbm.at[idxs[c]], vals[c], gsem.at[c]).wait()

        def reduce(i, _, c=c):
            jj = i * L

            def fstep(f, a):
                return a + vals[c][pl.ds(f * CS + jj, L)]

            outv[pl.ds(c * CS + jj, L)] = lax.fori_loop(0, F, fstep,
                                                        biasv[...], unroll=2)
            return 0

        lax.fori_loop(0, CS // L, reduce, 0)

    pltpu.sync_copy(outv, out_hbm.at[pl.ds(base, BPW)])


def kernel(x, fc_weight, bias):
    xt = x.astype(jnp.int32).T
    tabp = jnp.pad(fc_weight, ((0, VPAD - F * FIELD), (0, 0))).reshape(-1)
    bias16 = jnp.broadcast_to(bias.astype(jnp.float32), (L,))
    idx = _build_indices_sc(xt)
    out = _gather_reduce_sc(idx, bias16, tabp)
    return out.reshape(B, 1)


# speedup vs baseline: 1.0109x; 1.0109x over previous
import functools

import jax
import jax.numpy as jnp
from jax import lax
from jax.experimental import pallas as pl
from jax.experimental.pallas import tpu as pltpu
from jax.experimental.pallas import tpu_sc as plsc

B = 16384
F = 26
FIELD = 100000
VPAD = 2600960
NW = 32
BPW = B // NW
L = 16
NC = 4
CS = BPW // NC
CHUNK = F * CS

_mesh = plsc.VectorSubcoreMesh(core_axis_name="c", subcore_axis_name="s")


@functools.partial(
    pl.kernel,
    mesh=_mesh,
    out_type=jax.ShapeDtypeStruct((B,), jnp.float32),
    scratch_types=[
        pltpu.VMEM((NC, F, CS), jnp.int32),
        *[pltpu.VMEM((CHUNK,), jnp.int32) for _ in range(NC)],
        *[pltpu.VMEM((CHUNK,), jnp.float32) for _ in range(NC)],
        pltpu.VMEM((BPW,), jnp.float32),
        pltpu.VMEM((L,), jnp.float32),
        pltpu.SemaphoreType.DMA((NC,)),
        pltpu.SemaphoreType.DMA((NC,)),
    ],
)
def _features_linear_sc(xt_hbm, bias_hbm, tab_hbm, out_hbm,
                        xv, i0, i1, i2, i3, v0, v1, v2, v3,
                        outv, biasv, xsem, gsem):
    idxs = (i0, i1, i2, i3)
    vals = (v0, v1, v2, v3)
    wid = lax.axis_index("s") * 2 + lax.axis_index("c")
    base = wid * BPW

    for c in range(NC):
        pltpu.async_copy(xt_hbm.at[:, pl.ds(base + c * CS, CS)],
                         xv.at[c], xsem.at[c])

    for c in range(NC):
        pltpu.make_async_copy(xt_hbm.at[:, pl.ds(base + c * CS, CS)],
                              xv.at[c], xsem.at[c]).wait()

        def build(i, _, c=c):
            f = i // (CS // L)
            jj = (i % (CS // L)) * L
            idxs[c][pl.ds(i * L, L)] = xv[c, f, pl.ds(jj, L)] + f * FIELD
            return 0

        lax.fori_loop(0, CHUNK // L, build, 0, unroll=4)
        pltpu.async_copy(tab_hbm.at[idxs[c]], vals[c], gsem.at[c])

    pltpu.sync_copy(bias_hbm, biasv)

    for c in range(NC):
        pltpu.make_async_copy(tab_hbm.at[idxs[c]], vals[c], gsem.at[c]).wait()

        def reduce(i, _, c=c):
            jj = i * L

            def fstep(f, a):
                return a + vals[c][pl.ds(f * CS + jj, L)]

            outv[pl.ds(c * CS + jj, L)] = lax.fori_loop(0, F, fstep,
                                                        biasv[...], unroll=2)
            return 0

        lax.fori_loop(0, CS // L, reduce, 0)

    pltpu.sync_copy(outv, out_hbm.at[pl.ds(base, BPW)])


def kernel(x, fc_weight, bias):
    xt = x.astype(jnp.int32).T
    tabp = jnp.pad(fc_weight, ((0, VPAD - F * FIELD), (0, 0))).reshape(-1)
    bias16 = jnp.broadcast_to(bias.astype(jnp.float32), (L,))
    out = _features_linear_sc(xt, bias16, tabp)
    return out.reshape(B, 1)
